# baseline (speedup 1.0000x reference)
import functools

import jax
import jax.numpy as jnp
from jax import lax
from jax.experimental import pallas as pl
from jax.experimental.pallas import tpu as pltpu
from jax.experimental.pallas import tpu_sc as plsc

N_AGENTS = 1000000
N_BATCH = 16384
NC = 2
NS = 16
NW = NC * NS
L = 16

BPW = N_BATCH // NW
GCH = 128
NG = BPW // GCH

TC_CH = 65536
TC_GRID = -(-N_AGENTS // TC_CH)

_mesh = plsc.VectorSubcoreMesh(core_axis_name="c", subcore_axis_name="s")


@functools.partial(
    pl.kernel,
    mesh=_mesh,
    out_type=(
        jax.ShapeDtypeStruct((N_BATCH,), jnp.float32),
        jax.ShapeDtypeStruct((NC, L), jnp.float32),
    ),
    scratch_types=[
        pltpu.VMEM((NG, GCH), jnp.int32),
        pltpu.VMEM((BPW,), jnp.float32),
        pltpu.VMEM((BPW,), jnp.float32),
        pltpu.VMEM((BPW,), jnp.float32),
        pltpu.VMEM((BPW,), jnp.float32),
        pltpu.VMEM((L,), jnp.float32),
        pltpu.VMEM((NS * L,), jnp.float32),
        pltpu.VMEM_SHARED((NS * L,), jnp.float32),
        pltpu.SemaphoreType.DMA,
    ],
)
def _sc_doxastic(bel_hbm, gt_hbm, ids_hbm, log_hbm, cb_out, loss_out,
                 idx_v, gat_v, bel_v, gt_v, cb_v, part_v, all_v, shared,
                 sem_g):
    cid = lax.axis_index("c")
    sid = lax.axis_index("s")
    wid = cid * NS + sid
    b0 = wid * BPW

    for k in range(NG):
        pltpu.sync_copy(ids_hbm.at[pl.ds(b0 + k * GCH, GCH)], idx_v.at[k])
    gds = [
        pltpu.async_copy(
            log_hbm.at[idx_v.at[k]],
            gat_v.at[pl.ds(k * GCH, GCH)],
            sem_g,
        )
        for k in range(NG)
    ]
    pltpu.sync_copy(bel_hbm.at[pl.ds(b0, BPW)], bel_v)
    pltpu.sync_copy(gt_hbm.at[pl.ds(b0, BPW)], gt_v)
    for gd in gds:
        gd.wait()

    def body_b(j, carry):
        hl, cc = carry
        s = pl.ds(j * L, L)
        cal = 2.0 / (1.0 + jnp.exp(-gat_v[s]))
        cb = jnp.minimum(jnp.maximum(bel_v[s] * cal, 0.0), 1.0)
        cb_v[s] = cb
        g = gt_v[s]
        return hl + cb * (1.0 - g), cc + (1.0 - cb) * g

    zero = jnp.zeros((L,), jnp.float32)
    hl, cc = lax.fori_loop(0, BPW // L, body_b, (zero, zero))

    pltpu.sync_copy(cb_v, cb_out.at[pl.ds(b0, BPW)])

    part_v[...] = hl * (1.0 / N_BATCH) + cc * (0.5 / N_BATCH)
    pltpu.sync_copy(part_v, shared.at[pl.ds(sid * L, L)])
    plsc.subcore_barrier()

    @pl.when(sid == 0)
    def _():
        pltpu.sync_copy(shared, all_v)

        def body_r(s, acc):
            return acc + all_v[pl.ds(s * L, L)]

        acc = lax.fori_loop(0, NS, body_r, jnp.zeros((L,), jnp.float32))
        part_v[...] = acc
        pltpu.sync_copy(part_v, loss_out.at[cid])


def _tc_calreg_body(x_ref, o_ref):
    i = pl.program_id(0)

    @pl.when(i == 0)
    def _():
        o_ref[0] = 0.0

    t = jnp.exp(-jnp.abs(x_ref[...]))
    f = (1.0 - t) / (1.0 + t)
    pos = i * TC_CH + lax.iota(jnp.int32, TC_CH)
    f = jnp.where(pos < N_AGENTS, f, 0.0)
    o_ref[0] += jnp.sum(f)


_tc_calreg = pl.pallas_call(
    _tc_calreg_body,
    grid=(TC_GRID,),
    in_specs=[pl.BlockSpec((TC_CH,), lambda i: (i,))],
    out_specs=pl.BlockSpec(memory_space=pltpu.SMEM),
    out_shape=jax.ShapeDtypeStruct((1,), jnp.float32),
    compiler_params=pltpu.CompilerParams(
        dimension_semantics=("arbitrary",)),
)


def kernel(belief_strength, ground_truth, agent_ids, calibration_logits):
    ids = agent_ids.astype(jnp.int32)
    cb, loss_parts = _sc_doxastic(belief_strength, ground_truth, ids,
                                  calibration_logits)
    calreg_sum = _tc_calreg(calibration_logits)
    loss = jnp.sum(loss_parts) + (0.1 / N_AGENTS) * calreg_sum[0]
    return (loss, cb)


# speedup vs baseline: 1.1687x; 1.1687x over previous
import functools

import jax
import jax.numpy as jnp
from jax import lax
from jax.experimental import pallas as pl
from jax.experimental.pallas import tpu as pltpu
from jax.experimental.pallas import tpu_sc as plsc

N_AGENTS = 1000000
N_BATCH = 16384
NC = 2
NS = 16
NW = NC * NS
L = 16

BPW = N_BATCH // NW
GCH = 128
NG = BPW // GCH

TC_CH = 65536
TC_GRID = 10
SC_T0 = TC_GRID * TC_CH
VPT = (N_AGENTS - SC_T0 - 64) // (NW * L)
EPT = VPT * L
SC_EX0 = SC_T0 + NW * EPT
NEXTRA = (N_AGENTS - SC_EX0) // L
VU = 4
VMAIN = VPT - (VPT % VU)

_mesh = plsc.VectorSubcoreMesh(core_axis_name="c", subcore_axis_name="s")


def _q_of(x):
    return 1.0 / (1.0 + jnp.exp(jnp.minimum(x, -x)))


@functools.partial(
    pl.kernel,
    mesh=_mesh,
    out_type=(
        jax.ShapeDtypeStruct((N_BATCH,), jnp.float32),
        jax.ShapeDtypeStruct((NC, L), jnp.float32),
    ),
    scratch_types=[
        pltpu.VMEM((NG, GCH), jnp.int32),
        pltpu.VMEM((BPW,), jnp.float32),
        pltpu.VMEM((BPW,), jnp.float32),
        pltpu.VMEM((BPW,), jnp.float32),
        pltpu.VMEM((BPW,), jnp.float32),
        pltpu.VMEM((EPT,), jnp.float32),
        pltpu.VMEM((L,), jnp.float32),
        pltpu.VMEM((L,), jnp.float32),
        pltpu.VMEM((NS * L,), jnp.float32),
        pltpu.VMEM_SHARED((NS * L,), jnp.float32),
        pltpu.SemaphoreType.DMA,
        pltpu.SemaphoreType.DMA,
    ],
)
def _sc_doxastic(bel_hbm, gt_hbm, ids_hbm, log_hbm, cb_out, loss_out,
                 idx_v, gat_v, bel_v, gt_v, cb_v, chunk_v, extra_v, part_v,
                 all_v, shared, sem_g, sem_c):
    cid = lax.axis_index("c")
    sid = lax.axis_index("s")
    wid = cid * NS + sid
    b0 = wid * BPW

    cd = pltpu.async_copy(log_hbm.at[pl.ds(SC_T0 + wid * EPT, EPT)],
                          chunk_v, sem_c)
    for k in range(NG):
        pltpu.sync_copy(ids_hbm.at[pl.ds(b0 + k * GCH, GCH)], idx_v.at[k])
    gds = [
        pltpu.async_copy(
            log_hbm.at[idx_v.at[k]],
            gat_v.at[pl.ds(k * GCH, GCH)],
            sem_g,
        )
        for k in range(NG)
    ]
    pltpu.sync_copy(bel_hbm.at[pl.ds(b0, BPW)], bel_v)
    pltpu.sync_copy(gt_hbm.at[pl.ds(b0, BPW)], gt_v)
    eoff = SC_EX0 + (wid % NEXTRA) * L
    pltpu.sync_copy(log_hbm.at[pl.ds(eoff, L)], extra_v)

    cd.wait()

    def body_a(j, accs):
        a0, a1, a2, a3 = accs
        base = j * (VU * L)
        a0 = a0 + _q_of(chunk_v[pl.ds(base, L)])
        a1 = a1 + _q_of(chunk_v[pl.ds(base + L, L)])
        a2 = a2 + _q_of(chunk_v[pl.ds(base + 2 * L, L)])
        a3 = a3 + _q_of(chunk_v[pl.ds(base + 3 * L, L)])
        return a0, a1, a2, a3

    zero = jnp.zeros((L,), jnp.float32)
    a0, a1, a2, a3 = lax.fori_loop(0, VMAIN // VU, body_a,
                                   (zero, zero, zero, zero))
    qs = a0 + a1 + a2 + a3
    for v in range(VMAIN, VPT):
        qs = qs + _q_of(chunk_v[pl.ds(v * L, L)])
    emask = (wid < NEXTRA).astype(jnp.float32)
    qs = qs + 0.5 + (_q_of(extra_v[...]) - 0.5) * emask
    cr = 2.0 * qs - float(VPT + 1)

    for gd in gds:
        gd.wait()

    def body_b(j, carry):
        hl, cc = carry
        s = pl.ds(j * L, L)
        cal = 2.0 / (1.0 + jnp.exp(-gat_v[s]))
        cb = jnp.minimum(jnp.maximum(bel_v[s] * cal, 0.0), 1.0)
        cb_v[s] = cb
        g = gt_v[s]
        return hl + cb * (1.0 - g), cc + (1.0 - cb) * g

    hl, cc = lax.fori_loop(0, BPW // L, body_b, (zero, zero))

    pltpu.sync_copy(cb_v, cb_out.at[pl.ds(b0, BPW)])

    part_v[...] = (hl * (1.0 / N_BATCH) + cc * (0.5 / N_BATCH)
                   + cr * (0.1 / N_AGENTS))
    pltpu.sync_copy(part_v, shared.at[pl.ds(sid * L, L)])
    plsc.subcore_barrier()

    @pl.when(sid == 0)
    def _():
        pltpu.sync_copy(shared, all_v)

        def body_r(s, acc):
            return acc + all_v[pl.ds(s * L, L)]

        acc = lax.fori_loop(0, NS, body_r, jnp.zeros((L,), jnp.float32))
        part_v[...] = acc
        pltpu.sync_copy(part_v, loss_out.at[cid])


def _tc_calreg_body(x_ref, o_ref):
    i = pl.program_id(0)

    @pl.when(i == 0)
    def _():
        o_ref[0] = 0.0

    t = jnp.exp(-jnp.abs(x_ref[...]))
    o_ref[0] += jnp.sum((1.0 - t) / (1.0 + t))


_tc_calreg = pl.pallas_call(
    _tc_calreg_body,
    grid=(TC_GRID,),
    in_specs=[pl.BlockSpec((TC_CH,), lambda i: (i,))],
    out_specs=pl.BlockSpec(memory_space=pltpu.SMEM),
    out_shape=jax.ShapeDtypeStruct((1,), jnp.float32),
    compiler_params=pltpu.CompilerParams(
        dimension_semantics=("arbitrary",)),
)


def kernel(belief_strength, ground_truth, agent_ids, calibration_logits):
    ids = agent_ids.astype(jnp.int32)
    cb, loss_parts = _sc_doxastic(belief_strength, ground_truth, ids,
                                  calibration_logits)
    calreg_sum = _tc_calreg(calibration_logits)
    loss = jnp.sum(loss_parts) + (0.1 / N_AGENTS) * calreg_sum[0]
    return (loss, cb)


# speedup vs baseline: 1.1881x; 1.0166x over previous
import functools

import jax
import jax.numpy as jnp
from jax import lax
from jax.experimental import pallas as pl
from jax.experimental.pallas import tpu as pltpu
from jax.experimental.pallas import tpu_sc as plsc

N_AGENTS = 1000000
N_BATCH = 16384
NC = 2
NS = 16
NW = NC * NS
L = 16

BPW = N_BATCH // NW
GCH = 128
NG = BPW // GCH

TC_CH = 131072
TC_GRID = 5
SC_T0 = TC_GRID * TC_CH
VPT = (N_AGENTS - SC_T0 - 64) // (NW * L)
EPT = VPT * L
SC_EX0 = SC_T0 + NW * EPT
NEXTRA = (N_AGENTS - SC_EX0) // L
VU = 8
VMAIN = VPT - (VPT % VU)

_mesh = plsc.VectorSubcoreMesh(core_axis_name="c", subcore_axis_name="s")


def _q_of(x):
    return 1.0 / (1.0 + jnp.exp(jnp.minimum(x, -x)))


@functools.partial(
    pl.kernel,
    mesh=_mesh,
    out_type=(
        jax.ShapeDtypeStruct((N_BATCH,), jnp.float32),
        jax.ShapeDtypeStruct((NC, L), jnp.float32),
    ),
    scratch_types=[
        pltpu.VMEM((NG, GCH), jnp.int32),
        pltpu.VMEM((BPW,), jnp.float32),
        pltpu.VMEM((BPW,), jnp.float32),
        pltpu.VMEM((BPW,), jnp.float32),
        pltpu.VMEM((BPW,), jnp.float32),
        pltpu.VMEM((EPT,), jnp.float32),
        pltpu.VMEM((L,), jnp.float32),
        pltpu.VMEM((L,), jnp.float32),
        pltpu.VMEM((NS * L,), jnp.float32),
        pltpu.VMEM_SHARED((NS * L,), jnp.float32),
        pltpu.SemaphoreType.DMA,
        pltpu.SemaphoreType.DMA,
    ],
)
def _sc_doxastic(bel_hbm, gt_hbm, ids_hbm, log_hbm, cb_out, loss_out,
                 idx_v, gat_v, bel_v, gt_v, cb_v, chunk_v, extra_v, part_v,
                 all_v, shared, sem_g, sem_c):
    cid = lax.axis_index("c")
    sid = lax.axis_index("s")
    wid = cid * NS + sid
    b0 = wid * BPW

    cd = pltpu.async_copy(log_hbm.at[pl.ds(SC_T0 + wid * EPT, EPT)],
                          chunk_v, sem_c)
    for k in range(NG):
        pltpu.sync_copy(ids_hbm.at[pl.ds(b0 + k * GCH, GCH)], idx_v.at[k])
    gds = [
        pltpu.async_copy(
            log_hbm.at[idx_v.at[k]],
            gat_v.at[pl.ds(k * GCH, GCH)],
            sem_g,
        )
        for k in range(NG)
    ]
    pltpu.sync_copy(bel_hbm.at[pl.ds(b0, BPW)], bel_v)
    pltpu.sync_copy(gt_hbm.at[pl.ds(b0, BPW)], gt_v)
    eoff = SC_EX0 + (wid % NEXTRA) * L
    pltpu.sync_copy(log_hbm.at[pl.ds(eoff, L)], extra_v)

    cd.wait()

    def body_a(j, accs):
        base = j * (VU * L)
        return tuple(
            a + _q_of(chunk_v[pl.ds(base + u * L, L)])
            for u, a in enumerate(accs)
        )

    zero = jnp.zeros((L,), jnp.float32)
    accs = lax.fori_loop(0, VMAIN // VU, body_a, (zero,) * VU)
    qs = accs[0]
    for a in accs[1:]:
        qs = qs + a
    for v in range(VMAIN, VPT):
        qs = qs + _q_of(chunk_v[pl.ds(v * L, L)])
    emask = (wid < NEXTRA).astype(jnp.float32)
    qs = qs + 0.5 + (_q_of(extra_v[...]) - 0.5) * emask
    cr = 2.0 * qs - float(VPT + 1)

    for gd in gds:
        gd.wait()

    def body_b(j, carry):
        hl, cc = carry
        s = pl.ds(j * L, L)
        cal = 2.0 / (1.0 + jnp.exp(-gat_v[s]))
        cb = jnp.minimum(jnp.maximum(bel_v[s] * cal, 0.0), 1.0)
        cb_v[s] = cb
        g = gt_v[s]
        return hl + cb * (1.0 - g), cc + (1.0 - cb) * g

    hl, cc = lax.fori_loop(0, BPW // L, body_b, (zero, zero))

    pltpu.sync_copy(cb_v, cb_out.at[pl.ds(b0, BPW)])

    part_v[...] = (hl * (1.0 / N_BATCH) + cc * (0.5 / N_BATCH)
                   + cr * (0.1 / N_AGENTS))
    pltpu.sync_copy(part_v, shared.at[pl.ds(sid * L, L)])
    plsc.subcore_barrier()

    @pl.when(sid == 0)
    def _():
        pltpu.sync_copy(shared, all_v)

        def body_r(s, acc):
            return acc + all_v[pl.ds(s * L, L)]

        acc = lax.fori_loop(0, NS, body_r, jnp.zeros((L,), jnp.float32))
        part_v[...] = acc
        pltpu.sync_copy(part_v, loss_out.at[cid])


def _tc_calreg_body(x_ref, o_ref):
    i = pl.program_id(0)

    @pl.when(i == 0)
    def _():
        o_ref[0] = 0.0

    t = jnp.exp(-jnp.abs(x_ref[...]))
    o_ref[0] += jnp.sum((1.0 - t) / (1.0 + t))


_tc_calreg = pl.pallas_call(
    _tc_calreg_body,
    grid=(TC_GRID,),
    in_specs=[pl.BlockSpec((TC_CH,), lambda i: (i,))],
    out_specs=pl.BlockSpec(memory_space=pltpu.SMEM),
    out_shape=jax.ShapeDtypeStruct((1,), jnp.float32),
    compiler_params=pltpu.CompilerParams(
        dimension_semantics=("arbitrary",)),
)


def kernel(belief_strength, ground_truth, agent_ids, calibration_logits):
    ids = agent_ids.astype(jnp.int32)
    cb, loss_parts = _sc_doxastic(belief_strength, ground_truth, ids,
                                  calibration_logits)
    calreg_sum = _tc_calreg(calibration_logits)
    loss = jnp.sum(loss_parts) + (0.1 / N_AGENTS) * calreg_sum[0]
    return (loss, cb)
